# baseline (speedup 1.0000x reference)
import jax
import jax.numpy as jnp
from jax import lax
from jax.experimental import pallas as pl

_N = 16384
_C = 1000
_B = 2048
_W = 128
_NFULL = 7
_TAIL = _C - _NFULL * _W


def _body(t_ref, x_ref, o_ref):
    i = pl.program_id(0)

    @pl.when(i == 0)
    def _init():
        o_ref[...] = jnp.zeros((1, 1), jnp.float32)

    m = x_ref[:, 0:_W]
    cb = jnp.zeros((_B, _W), jnp.int32)
    for k in range(1, _NFULL):
        v = x_ref[:, k * _W:(k + 1) * _W]
        upd = v > m
        m = jnp.where(upd, v, m)
        cb = jnp.where(upd, k * _W, cb)
    v7 = x_ref[:, _NFULL * _W:_C]

    gm = jnp.maximum(
        jnp.max(m, axis=1, keepdims=True),
        jnp.max(v7, axis=1, keepdims=True),
    )
    lane = lax.broadcasted_iota(jnp.int32, (_B, _W), 1)
    first_main = jnp.min(jnp.where(m == gm, cb + lane, _C), axis=1)
    lane7 = lax.broadcasted_iota(jnp.int32, (_B, _TAIL), 1) + _NFULL * _W
    first_tail = jnp.min(jnp.where(v7 == gm, lane7, _C), axis=1)
    first = jnp.minimum(first_main, first_tail)

    t = t_ref[0, 0, :]
    cnt = jnp.sum((first == t).astype(jnp.float32)).reshape(1, 1)
    o_ref[...] = o_ref[...] + cnt

    @pl.when(i == pl.num_programs(0) - 1)
    def _final():
        o_ref[...] = o_ref[...] * (1.0 / _N)


def kernel(true, logits):
    grid = _N // _B
    t3 = true.reshape(grid, 1, _B).astype(jnp.int32)
    out = pl.pallas_call(
        _body,
        grid=(grid,),
        in_specs=[
            pl.BlockSpec((1, 1, _B), lambda i: (i, 0, 0)),
            pl.BlockSpec((_B, _C), lambda i: (i, 0)),
        ],
        out_specs=pl.BlockSpec((1, 1), lambda i: (0, 0)),
        out_shape=jax.ShapeDtypeStruct((1, 1), jnp.float32),
    )(t3, logits)
    return out[0, 0]
